# Initial kernel scaffold; baseline (speedup 1.0000x reference)
import jax
import jax.numpy as jnp
from jax.experimental import pallas as pl


def kernel(h_subgraph, subgraph_idx, W_fc0, b_fc0, W_sum0, b_sum0, W_fc1, b_fc1, W_sum1, b_sum1, Wf1, bf1, Wf2, bf2):
    raise NotImplementedError("write your pallas kernel here")


# speedup vs baseline: 1.4398x; 1.4398x over previous
import jax
import jax.numpy as jnp
from jax.experimental import pallas as pl
from jax.experimental.pallas import tpu as pltpu

N_ROWS = 32768
N_GRAPHS = 1024
IN_DIM = 300
K0 = 304
CH = 256
BLK = 256
NB = N_ROWS // BLK


def _segsum_body(idx_ref, x_ref, p_ref, c_ref):
    i = pl.program_id(0)

    @pl.when(i == 0)
    def _():
        p_ref[...] = jnp.zeros_like(p_ref)
        c_ref[...] = jnp.zeros_like(c_ref)

    idxv = idx_ref[0, 0, :]
    iota = jax.lax.broadcasted_iota(jnp.int32, (BLK, N_GRAPHS), 1)
    oh = jnp.where(iota == idxv[:, None], 1.0, 0.0)
    p_ref[0] += jax.lax.dot_general(
        oh, x_ref[...], (((0,), (0,)), ((), ())),
        preferred_element_type=jnp.float32)
    c_ref[0] += jax.lax.dot_general(
        oh, jnp.ones((BLK, 16), jnp.float32), (((0,), (0,)), ((), ())),
        preferred_element_type=jnp.float32)


def _tc_segsum(x, idx3, want_counts):
    k = x.shape[1]
    return pl.pallas_call(
        _segsum_body,
        grid=(NB,),
        in_specs=[
            pl.BlockSpec((1, 1, BLK), lambda i: (i, 0, 0)),
            pl.BlockSpec((BLK, k), lambda i: (i, 0)),
        ],
        out_specs=[
            pl.BlockSpec((1, N_GRAPHS, k), lambda i: (0, 0, 0)),
            pl.BlockSpec((1, N_GRAPHS, 16), lambda i: (0, 0, 0)),
        ],
        out_shape=[
            jax.ShapeDtypeStruct((1, N_GRAPHS, k), jnp.float32),
            jax.ShapeDtypeStruct((1, N_GRAPHS, 16), jnp.float32),
        ],
    )(idx3, x)


def _layer_body(idx_ref, h_ref, x2_ref, w_ref, b_ref, o_ref):
    idxv = idx_ref[0, 0, :]
    iota = jax.lax.broadcasted_iota(jnp.int32, (BLK, N_GRAPHS), 1)
    oh = jnp.where(iota == idxv[:, None], 1.0, 0.0)
    x2g = jnp.dot(oh, x2_ref[...], preferred_element_type=jnp.float32)
    x1 = jnp.dot(h_ref[...], w_ref[...], preferred_element_type=jnp.float32)
    s = x1 + x2g + b_ref[...]
    o_ref[...] = jnp.where(s > 0.0, s, jnp.exp(jnp.minimum(s, 0.0)) - 1.0)


def _layer(h, idx3, x2, w, bias):
    k = h.shape[1]
    return pl.pallas_call(
        _layer_body,
        grid=(NB,),
        in_specs=[
            pl.BlockSpec((1, 1, BLK), lambda i: (i, 0, 0)),
            pl.BlockSpec((BLK, k), lambda i: (i, 0)),
            pl.BlockSpec((N_GRAPHS, CH), lambda i: (0, 0)),
            pl.BlockSpec((k, CH), lambda i: (0, 0)),
            pl.BlockSpec((1, CH), lambda i: (0, 0)),
        ],
        out_specs=pl.BlockSpec((BLK, CH), lambda i: (i, 0)),
        out_shape=jax.ShapeDtypeStruct((N_ROWS, CH), jnp.float32),
    )(idx3, h, x2, w, bias)


def _inv_counts(c_ref):
    cnt = jnp.sum(c_ref[...], axis=0)[:, 0:1]
    return 1.0 / jnp.maximum(cnt, 1.0)


def _x2_body(p_ref, c_ref, w_ref, o_ref):
    m = jnp.sum(p_ref[...], axis=0) * _inv_counts(c_ref)
    o_ref[...] = jnp.dot(m, w_ref[...], preferred_element_type=jnp.float32)


def _x2_small(p, c, w):
    k = p.shape[2]
    return pl.pallas_call(
        _x2_body,
        out_shape=jax.ShapeDtypeStruct((N_GRAPHS, CH), jnp.float32),
    )(p, c, w)


def _final_body(p_ref, c_ref, w1_ref, b1_ref, w2_ref, b2_ref, o_ref):
    m = jnp.sum(p_ref[...], axis=0) * _inv_counts(c_ref)
    t = jnp.dot(m, w1_ref[...], preferred_element_type=jnp.float32) + b1_ref[...]
    t = jnp.maximum(t, 0.0)
    o_ref[...] = jnp.dot(t, w2_ref[...], preferred_element_type=jnp.float32) + b2_ref[...]


def _final(p, c, w1, b1, w2, b2):
    nt = w2.shape[1]
    return pl.pallas_call(
        _final_body,
        out_shape=jax.ShapeDtypeStruct((N_GRAPHS, nt), jnp.float32),
    )(p, c, w1, b1, w2, b2)


def kernel(h_subgraph, subgraph_idx, W_fc0, b_fc0, W_sum0, b_sum0,
           W_fc1, b_fc1, W_sum1, b_sum1, Wf1, bf1, Wf2, bf2):
    idx = subgraph_idx.astype(jnp.int32)
    idx3 = idx.reshape(NB, 1, BLK)

    h = jnp.pad(h_subgraph, ((0, 0), (0, K0 - IN_DIM)))
    W0 = jnp.pad(W_fc0, ((0, K0 - IN_DIM), (0, 0)))
    Ws0 = jnp.pad(W_sum0, ((0, K0 - IN_DIM), (0, 0)))

    p0, c0 = _tc_segsum(h, idx3, True)
    x2_0 = _x2_small(p0, c0, Ws0)
    bias0 = (b_fc0 + b_sum0).reshape(1, CH)
    h1 = _layer(h, idx3, x2_0, W0, bias0)

    p1, _ = _tc_segsum(h1, idx3, False)
    x2_1 = _x2_small(p1, c0, W_sum1)
    bias1 = (b_fc1 + b_sum1).reshape(1, CH)
    h2 = _layer(h1, idx3, x2_1, W_fc1, bias1)

    p2, _ = _tc_segsum(h2, idx3, False)
    return _final(p2, c0, Wf1, bf1.reshape(1, -1), Wf2, bf2.reshape(1, -1))


# speedup vs baseline: 1.5996x; 1.1110x over previous
import functools

import jax
import jax.numpy as jnp
from jax import lax
from jax.experimental import pallas as pl
from jax.experimental.pallas import tpu as pltpu
from jax.experimental.pallas import tpu_sc as plsc

N_ROWS = 32768
N_GRAPHS = 1024
IN_DIM = 300
K0 = 384
CH = 256
BLK = 256
NB = N_ROWS // BLK

NC = 2
NS = 16
RPW = N_ROWS // (NC * NS)
CHUNK = 128
NCHUNK = RPW // CHUNK
GPS = N_GRAPHS // NS


def _sc_segsum_body(want_counts, d, x_hbm, idx_hbm, p_hbm, c_hbm,
                    dbuf, ibuf, ones, zbuf, zcnt, acc, cntacc):
    c = lax.axis_index("c")
    s = lax.axis_index("s")
    wid = s * NC + c
    d16 = d // 16

    def _fill(t, _):
        r = t // d16
        col = (t % d16) * 16
        zbuf[r, pl.ds(col, 16)] = jnp.zeros((16,), jnp.float32)
        return 0

    lax.fori_loop(0, GPS * d16, _fill, 0)
    if want_counts:
        def _fill1(r, _):
            ones[r, :] = jnp.ones((16,), jnp.float32)
            return 0

        lax.fori_loop(0, CHUNK, _fill1, 0)

        def _fill0(r, _):
            zcnt[r, :] = jnp.zeros((16,), jnp.float32)
            return 0

        lax.fori_loop(0, GPS, _fill0, 0)
        pltpu.sync_copy(zcnt, cntacc.at[pl.ds(s * GPS, GPS)])
    pltpu.sync_copy(zbuf, acc.at[pl.ds(s * GPS, GPS)])
    plsc.subcore_barrier()

    for j in range(NCHUNK):
        base = wid * RPW + j * CHUNK
        pltpu.sync_copy(idx_hbm.at[pl.ds(base, CHUNK)], ibuf)
        pltpu.sync_copy(x_hbm.at[pl.ds(base, CHUNK)], dbuf)
        pltpu.sync_copy(dbuf, acc.at[ibuf], add=True)
        if want_counts:
            pltpu.sync_copy(ones, cntacc.at[ibuf], add=True)

    plsc.subcore_barrier()
    pltpu.sync_copy(acc.at[pl.ds(s * GPS, GPS)], p_hbm.at[c, pl.ds(s * GPS, GPS)])
    if want_counts:
        pltpu.sync_copy(cntacc.at[pl.ds(s * GPS, GPS)],
                        c_hbm.at[c, pl.ds(s * GPS, GPS)])


def _sc_segsum(x, idx, want_counts):
    d = x.shape[1]
    mesh = plsc.VectorSubcoreMesh(core_axis_name="c", subcore_axis_name="s")
    out_type = [
        jax.ShapeDtypeStruct((NC, N_GRAPHS, d), jnp.float32),
        jax.ShapeDtypeStruct((NC, N_GRAPHS, 16), jnp.float32),
    ]
    scratch = [
        pltpu.VMEM((CHUNK, d), jnp.float32),
        pltpu.VMEM((CHUNK,), jnp.int32),
        pltpu.VMEM((CHUNK, 16), jnp.float32),
        pltpu.VMEM((GPS, d), jnp.float32),
        pltpu.VMEM((GPS, 16), jnp.float32),
        pltpu.VMEM_SHARED((N_GRAPHS, d), jnp.float32),
        pltpu.VMEM_SHARED((N_GRAPHS, 16), jnp.float32),
    ]
    fn = pl.kernel(
        functools.partial(_sc_segsum_body, want_counts, d),
        out_type=out_type, mesh=mesh, scratch_types=scratch,
        compiler_params=pltpu.CompilerParams(use_tc_tiling_on_sc=False),
        name=f"sc_segsum_{d}_{int(want_counts)}",
    )
    return fn(x, idx)


def _layer_body(idx_ref, h_ref, x2_ref, w_ref, b_ref, o_ref):
    idxv = idx_ref[0, 0, :]
    iota = jax.lax.broadcasted_iota(jnp.int32, (BLK, N_GRAPHS), 1)
    oh = jnp.where(iota == idxv[:, None], 1.0, 0.0)
    x2g = jnp.dot(oh, x2_ref[...], preferred_element_type=jnp.float32)
    x1 = jnp.dot(h_ref[...], w_ref[...], preferred_element_type=jnp.float32)
    s = x1 + x2g + b_ref[...]
    o_ref[...] = jnp.where(s > 0.0, s, jnp.exp(jnp.minimum(s, 0.0)) - 1.0)


def _layer(h, idx3, x2, w, bias):
    k = h.shape[1]
    return pl.pallas_call(
        _layer_body,
        grid=(NB,),
        in_specs=[
            pl.BlockSpec((1, 1, BLK), lambda i: (i, 0, 0)),
            pl.BlockSpec((BLK, k), lambda i: (i, 0)),
            pl.BlockSpec((N_GRAPHS, CH), lambda i: (0, 0)),
            pl.BlockSpec((k, CH), lambda i: (0, 0)),
            pl.BlockSpec((1, CH), lambda i: (0, 0)),
        ],
        out_specs=pl.BlockSpec((BLK, CH), lambda i: (i, 0)),
        out_shape=jax.ShapeDtypeStruct((N_ROWS, CH), jnp.float32),
    )(idx3, h, x2, w, bias)


def _inv_counts(c_ref):
    cnt = jnp.sum(c_ref[...], axis=0)[:, 0:1]
    return 1.0 / jnp.maximum(cnt, 1.0)


def _x2_body(p_ref, c_ref, w_ref, o_ref):
    m = jnp.sum(p_ref[...], axis=0) * _inv_counts(c_ref)
    o_ref[...] = jnp.dot(m, w_ref[...], preferred_element_type=jnp.float32)


def _x2_small(p, c, w):
    k = p.shape[2]
    return pl.pallas_call(
        _x2_body,
        out_shape=jax.ShapeDtypeStruct((N_GRAPHS, CH), jnp.float32),
    )(p, c, w)


def _final_body(p_ref, c_ref, w1_ref, b1_ref, w2_ref, b2_ref, o_ref):
    m = jnp.sum(p_ref[...], axis=0) * _inv_counts(c_ref)
    t = jnp.dot(m, w1_ref[...], preferred_element_type=jnp.float32) + b1_ref[...]
    t = jnp.maximum(t, 0.0)
    o_ref[...] = jnp.dot(t, w2_ref[...], preferred_element_type=jnp.float32) + b2_ref[...]


def _final(p, c, w1, b1, w2, b2):
    nt = w2.shape[1]
    return pl.pallas_call(
        _final_body,
        out_shape=jax.ShapeDtypeStruct((N_GRAPHS, nt), jnp.float32),
    )(p, c, w1, b1, w2, b2)


def kernel(h_subgraph, subgraph_idx, W_fc0, b_fc0, W_sum0, b_sum0,
           W_fc1, b_fc1, W_sum1, b_sum1, Wf1, bf1, Wf2, bf2):
    idx = subgraph_idx.astype(jnp.int32)
    idx3 = idx.reshape(NB, 1, BLK)

    h = jnp.pad(h_subgraph, ((0, 0), (0, K0 - IN_DIM)))
    W0 = jnp.pad(W_fc0, ((0, K0 - IN_DIM), (0, 0)))
    Ws0 = jnp.pad(W_sum0, ((0, K0 - IN_DIM), (0, 0)))

    p0, c0 = _sc_segsum(h, idx, True)
    x2_0 = _x2_small(p0, c0, Ws0)
    bias0 = (b_fc0 + b_sum0).reshape(1, CH)
    h1 = _layer(h, idx3, x2_0, W0, bias0)

    p1, _ = _sc_segsum(h1, idx, False)
    x2_1 = _x2_small(p1, c0, W_sum1)
    bias1 = (b_fc1 + b_sum1).reshape(1, CH)
    h2 = _layer(h1, idx3, x2_1, W_fc1, bias1)

    p2, _ = _sc_segsum(h2, idx, False)
    return _final(p2, c0, Wf1, bf1.reshape(1, -1), Wf2, bf2.reshape(1, -1))


# speedup vs baseline: 1.6871x; 1.0547x over previous
import functools

import jax
import jax.numpy as jnp
from jax import lax
from jax.experimental import pallas as pl
from jax.experimental.pallas import tpu as pltpu
from jax.experimental.pallas import tpu_sc as plsc

N_ROWS = 32768
N_GRAPHS = 1024
IN_DIM = 300
K0 = 384
CH = 256
BLK = 256
NB = N_ROWS // BLK

NC = 2
NS = 16
RPW = N_ROWS // (NC * NS)
CHUNK = 64
NCHUNK = RPW // CHUNK
GPS = N_GRAPHS // NS


def _sc_segsum_body(want_counts, d, x_hbm, idx_hbm, p_hbm, c_hbm,
                    dbufa, dbufb, ibuf, ones, zbuf, zcnt, acc, cntacc,
                    sema, semb):
    c = lax.axis_index("c")
    s = lax.axis_index("s")
    wid = s * NC + c
    d16 = d // 16

    def _fill(t, _):
        r = t // d16
        col = (t % d16) * 16
        zbuf[r, pl.ds(col, 16)] = jnp.zeros((16,), jnp.float32)
        return 0

    lax.fori_loop(0, GPS * d16, _fill, 0)
    if want_counts:
        def _fill1(r, _):
            ones[r, :] = jnp.ones((16,), jnp.float32)
            return 0

        lax.fori_loop(0, CHUNK, _fill1, 0)

        def _fill0(r, _):
            zcnt[r, :] = jnp.zeros((16,), jnp.float32)
            return 0

        lax.fori_loop(0, GPS, _fill0, 0)
        pltpu.sync_copy(zcnt, cntacc.at[pl.ds(s * GPS, GPS)])
    pltpu.sync_copy(zbuf, acc.at[pl.ds(s * GPS, GPS)])

    pltpu.sync_copy(idx_hbm.at[pl.ds(wid * NCHUNK, NCHUNK)], ibuf)
    plsc.subcore_barrier()

    bufs = (dbufa, dbufb)
    sems = (sema, semb)
    row0 = wid * RPW
    pending = pltpu.async_copy(x_hbm.at[pl.ds(row0, CHUNK)], bufs[0], sems[0])
    for j in range(NCHUNK):
        cur = bufs[j % 2]
        pending.wait()
        if j + 1 < NCHUNK:
            pending = pltpu.async_copy(
                x_hbm.at[pl.ds(row0 + (j + 1) * CHUNK, CHUNK)],
                bufs[(j + 1) % 2], sems[(j + 1) % 2])
        pltpu.sync_copy(cur, acc.at[ibuf.at[j]], add=True)
        if want_counts:
            pltpu.sync_copy(ones, cntacc.at[ibuf.at[j]], add=True)

    plsc.subcore_barrier()
    pltpu.sync_copy(acc.at[pl.ds(s * GPS, GPS)], p_hbm.at[c, pl.ds(s * GPS, GPS)])
    if want_counts:
        pltpu.sync_copy(cntacc.at[pl.ds(s * GPS, GPS)],
                        c_hbm.at[c, pl.ds(s * GPS, GPS)])


def _sc_segsum(x, idx2d, want_counts):
    d = x.shape[1]
    mesh = plsc.VectorSubcoreMesh(core_axis_name="c", subcore_axis_name="s")
    out_type = [
        jax.ShapeDtypeStruct((NC, N_GRAPHS, d), jnp.float32),
        jax.ShapeDtypeStruct((NC, N_GRAPHS, 16), jnp.float32),
    ]
    scratch = [
        pltpu.VMEM((CHUNK, d), jnp.float32),
        pltpu.VMEM((CHUNK, d), jnp.float32),
        pltpu.VMEM((NCHUNK, CHUNK), jnp.int32),
        pltpu.VMEM((CHUNK, 16), jnp.float32),
        pltpu.VMEM((GPS, d), jnp.float32),
        pltpu.VMEM((GPS, 16), jnp.float32),
        pltpu.VMEM_SHARED((N_GRAPHS, d), jnp.float32),
        pltpu.VMEM_SHARED((N_GRAPHS, 16), jnp.float32),
        pltpu.SemaphoreType.DMA,
        pltpu.SemaphoreType.DMA,
    ]
    fn = pl.kernel(
        functools.partial(_sc_segsum_body, want_counts, d),
        out_type=out_type, mesh=mesh, scratch_types=scratch,
        compiler_params=pltpu.CompilerParams(use_tc_tiling_on_sc=False),
        name=f"sc_segsum_{d}_{int(want_counts)}",
    )
    return fn(x, idx2d)


def _layer_body(idx_ref, h_ref, x2_ref, w_ref, b_ref, o_ref):
    idxv = idx_ref[0, 0, :]
    iota = jax.lax.broadcasted_iota(jnp.int32, (BLK, N_GRAPHS), 1)
    oh = jnp.where(iota == idxv[:, None], 1.0, 0.0).astype(jnp.bfloat16)
    x2g = jnp.dot(oh, x2_ref[...], preferred_element_type=jnp.float32)
    x1 = jnp.dot(h_ref[...].astype(jnp.bfloat16), w_ref[...],
                 preferred_element_type=jnp.float32)
    s = x1 + x2g + b_ref[...]
    o_ref[...] = jnp.where(s > 0.0, s, jnp.exp(jnp.minimum(s, 0.0)) - 1.0)


def _layer(h, idx3, x2, w, bias):
    k = h.shape[1]
    return pl.pallas_call(
        _layer_body,
        grid=(NB,),
        in_specs=[
            pl.BlockSpec((1, 1, BLK), lambda i: (i, 0, 0)),
            pl.BlockSpec((BLK, k), lambda i: (i, 0)),
            pl.BlockSpec((N_GRAPHS, CH), lambda i: (0, 0)),
            pl.BlockSpec((k, CH), lambda i: (0, 0)),
            pl.BlockSpec((1, CH), lambda i: (0, 0)),
        ],
        out_specs=pl.BlockSpec((BLK, CH), lambda i: (i, 0)),
        out_shape=jax.ShapeDtypeStruct((N_ROWS, CH), jnp.float32),
    )(idx3, h, x2, w, bias)


def _inv_counts(c_ref):
    cnt = jnp.sum(c_ref[...], axis=0)[:, 0:1]
    return 1.0 / jnp.maximum(cnt, 1.0)


def _x2_body(p_ref, c_ref, w_ref, o_ref):
    m = jnp.sum(p_ref[...], axis=0) * _inv_counts(c_ref)
    o_ref[...] = jnp.dot(m, w_ref[...],
                         preferred_element_type=jnp.float32).astype(jnp.bfloat16)


def _x2_small(p, c, w):
    return pl.pallas_call(
        _x2_body,
        out_shape=jax.ShapeDtypeStruct((N_GRAPHS, CH), jnp.bfloat16),
    )(p, c, w)


def _final_body(p_ref, c_ref, w1_ref, b1_ref, w2_ref, b2_ref, o_ref):
    m = jnp.sum(p_ref[...], axis=0) * _inv_counts(c_ref)
    t = jnp.dot(m, w1_ref[...], preferred_element_type=jnp.float32) + b1_ref[...]
    t = jnp.maximum(t, 0.0)
    o_ref[...] = jnp.dot(t, w2_ref[...], preferred_element_type=jnp.float32) + b2_ref[...]


def _final(p, c, w1, b1, w2, b2):
    nt = w2.shape[1]
    return pl.pallas_call(
        _final_body,
        out_shape=jax.ShapeDtypeStruct((N_GRAPHS, nt), jnp.float32),
    )(p, c, w1, b1, w2, b2)


def kernel(h_subgraph, subgraph_idx, W_fc0, b_fc0, W_sum0, b_sum0,
           W_fc1, b_fc1, W_sum1, b_sum1, Wf1, bf1, Wf2, bf2):
    idx = subgraph_idx.astype(jnp.int32)
    idx3 = idx.reshape(NB, 1, BLK)
    idx2d = idx.reshape(N_ROWS // CHUNK, CHUNK)

    h = jnp.pad(h_subgraph, ((0, 0), (0, K0 - IN_DIM)))
    W0 = jnp.pad(W_fc0, ((0, K0 - IN_DIM), (0, 0))).astype(jnp.bfloat16)
    Ws0 = jnp.pad(W_sum0, ((0, K0 - IN_DIM), (0, 0)))

    p0, c0 = _sc_segsum(h, idx2d, True)
    x2_0 = _x2_small(p0, c0, Ws0)
    bias0 = (b_fc0 + b_sum0).reshape(1, CH)
    h1 = _layer(h, idx3, x2_0, W0, bias0)

    p1, _ = _sc_segsum(h1, idx2d, False)
    x2_1 = _x2_small(p1, c0, W_sum1)
    bias1 = (b_fc1 + b_sum1).reshape(1, CH)
    h2 = _layer(h1, idx3, x2_1, W_fc1.astype(jnp.bfloat16), bias1)

    p2, _ = _sc_segsum(h2, idx2d, False)
    return _final(p2, c0, Wf1, bf1.reshape(1, -1), Wf2, bf2.reshape(1, -1))
